# Initial kernel scaffold; baseline (speedup 1.0000x reference)
import jax
import jax.numpy as jnp
from jax.experimental import pallas as pl


def kernel(indices, table):
    raise NotImplementedError("write your pallas kernel here")


# speedup vs baseline: 3.5483x; 3.5483x over previous
import functools

import jax
import jax.numpy as jnp
from jax import lax
from jax.experimental import pallas as pl
from jax.experimental.pallas import tpu as pltpu
from jax.experimental.pallas import tpu_sc as plsc

VOCAB = 100000
EMBED_DIM = 64
BATCH = 4096
MAX_LEN = 200

N_ROWS = BATCH * MAX_LEN
NUM_WORKERS = 32
PER_WORKER = N_ROWS // NUM_WORKERS
CHUNK = 128
N_CHUNKS = PER_WORKER // CHUNK

_MESH = plsc.VectorSubcoreMesh(core_axis_name="c", subcore_axis_name="s")


@functools.partial(
    pl.kernel,
    mesh=_MESH,
    out_type=jax.ShapeDtypeStruct((N_ROWS, EMBED_DIM), jnp.float32),
    scratch_types=[
        pltpu.VMEM((N_CHUNKS, CHUNK), jnp.int32),
        pltpu.VMEM((CHUNK, EMBED_DIM), jnp.float32),
        pltpu.SemaphoreType.DMA,
    ],
    compiler_params=pltpu.CompilerParams(use_tc_tiling_on_sc=False),
)
def _gather_rows(idx_hbm, table_hbm, out_hbm, idx_v, rows_v, sem):
    wid = lax.axis_index("s") * 2 + lax.axis_index("c")
    base = wid * PER_WORKER
    pltpu.sync_copy(idx_hbm.at[wid], idx_v)

    def body(c, carry):
        pltpu.async_copy(table_hbm.at[idx_v.at[c]], rows_v, sem).wait()
        pltpu.sync_copy(rows_v, out_hbm.at[pl.ds(base + c * CHUNK, CHUNK)])
        return carry

    lax.fori_loop(0, N_CHUNKS, body, 0)


def kernel(indices, table):
    idx3 = indices.reshape(NUM_WORKERS, N_CHUNKS, CHUNK)
    out = _gather_rows(idx3, table)
    return out.reshape(BATCH, MAX_LEN, EMBED_DIM)
